# Initial kernel scaffold; baseline (speedup 1.0000x reference)
import jax
import jax.numpy as jnp
from jax.experimental import pallas as pl


def kernel(x, table, W, b):
    raise NotImplementedError("write your pallas kernel here")


# speedup vs baseline: 2.4032x; 2.4032x over previous
import functools

import jax
import jax.numpy as jnp
from jax import lax
from jax.experimental import pallas as pl
from jax.experimental.pallas import tpu as pltpu
from jax.experimental.pallas import tpu_sc as plsc

NC = 2
NS = 16
NW = NC * NS

CH = 128
K = 2


def _pool_sc(table, x2d, dstb2d, zeros, *, B, L, DIM):
    spt = B // NW
    rpt = spt * L
    nj = rpt // CH
    ng = nj // K

    mesh = plsc.VectorSubcoreMesh(core_axis_name="c", subcore_axis_name="s")

    @functools.partial(
        pl.kernel,
        mesh=mesh,
        compiler_params=pltpu.CompilerParams(use_tc_tiling_on_sc=False),
        out_type=jax.ShapeDtypeStruct((B, DIM), jnp.float32),
        scratch_types=[
            pltpu.VMEM((nj, CH), jnp.int32),
            pltpu.VMEM((nj, CH), jnp.int32),
            pltpu.VMEM((K * CH, DIM), jnp.float32),
            pltpu.VMEM((K * CH, DIM), jnp.float32),
            pltpu.VMEM_SHARED((NS * (B // NW), DIM), jnp.float32),
            pltpu.SemaphoreType.DMA,
            pltpu.SemaphoreType.DMA,
        ],
    )
    def pool(table_h, x_h, dstb_h, zeros_h, out_h, idx_all, dstb, rows0,
             rows1, acc, sem0, sem1):
        c = lax.axis_index("c")
        s = lax.axis_index("s")
        wid = c * NS + s
        row0 = s * spt

        pltpu.sync_copy(x_h.at[pl.ds(wid * nj, nj)], idx_all)
        pltpu.sync_copy(dstb_h.at[pl.ds(s * nj, nj)], dstb)
        pltpu.sync_copy(zeros_h, rows0)
        for z in range(spt // (K * CH)):
            pltpu.sync_copy(rows0, acc.at[pl.ds(row0 + z * K * CH, K * CH)])

        rows = (rows0, rows1)
        sems = (sem0, sem1)

        def fire(g, h):
            for k in range(K):
                j = g * K + k
                pltpu.async_copy(
                    table_h.at[idx_all.at[j]],
                    rows[h].at[pl.ds(k * CH, CH)],
                    sems[h],
                )

        fire(0, 0)

        def outer(gg, carry):
            for hh in range(2):
                g = gg * 2 + hh

                @pl.when(g + 1 < ng)
                def _():
                    fire(g + 1, (hh + 1) % 2)

                pltpu.make_async_copy(
                    table_h.at[pl.ds(0, K * CH)], rows[hh], sems[hh]
                ).wait()

                for k in range(K):
                    j = g * K + k
                    pltpu.sync_copy(
                        rows[hh].at[pl.ds(k * CH, CH)],
                        acc.at[dstb.at[j]],
                        add=True,
                    )
            return carry

        lax.fori_loop(0, ng // 2, outer, 0)

        pltpu.sync_copy(acc.at[pl.ds(row0, spt)], out_h.at[pl.ds(wid * spt, spt)])

    return pool(table, x2d, dstb2d, zeros)


def _classify_tc(pooled, W, b2, *, B, L, DIM, NCLS, BM):
    inv_l = 1.0 / L

    def body(p_ref, w_ref, b_ref, o_ref):
        o_ref[...] = (
            jnp.dot(p_ref[...], w_ref[...], preferred_element_type=jnp.float32)
            * inv_l
            + b_ref[...]
        )

    return pl.pallas_call(
        body,
        grid=(B // BM,),
        in_specs=[
            pl.BlockSpec((BM, DIM), lambda i: (i, 0)),
            pl.BlockSpec((DIM, NCLS), lambda i: (0, 0)),
            pl.BlockSpec((1, NCLS), lambda i: (0, 0)),
        ],
        out_specs=pl.BlockSpec((BM, NCLS), lambda i: (i, 0)),
        out_shape=jax.ShapeDtypeStruct((B, NCLS), jnp.float32),
    )(pooled, W, b2)


def kernel(x, table, W, b):
    B, L = x.shape
    DIM = table.shape[1]
    NCLS = W.shape[1]

    x2d = x.astype(jnp.int32).reshape(B * L // CH, CH)
    rpt = B // NW * L
    dstb2d = (jnp.arange(NS * rpt, dtype=jnp.int32) // L).reshape(
        NS * rpt // CH, CH)
    zeros = jnp.zeros((K * CH, DIM), jnp.float32)
    pooled = _pool_sc(table, x2d, dstb2d, zeros, B=B, L=L, DIM=DIM)
    return _classify_tc(pooled, W, b.reshape(1, NCLS),
                        B=B, L=L, DIM=DIM, NCLS=NCLS, BM=1024)
